# baseline (speedup 1.0000x reference)
import functools

import jax
import jax.numpy as jnp
from jax.experimental import pallas as pl
from jax.experimental.pallas import tpu as pltpu

NUM_GRAPHS = 128


def _heads_body(g_ref, finger_ref, fcgW, fcgb, fp1W, fp1b, fp2W, fp2b,
                h1W, h1b, h2W, h2b, h3W, h3b, h4W, h4b, out_ref):
    g = g_ref[...]
    gf = jax.nn.relu(jnp.dot(g, fcgW[...], preferred_element_type=jnp.float32) + fcgb[...])
    fp = jax.nn.relu(jnp.dot(finger_ref[...], fp1W[...], preferred_element_type=jnp.float32) + fp1b[...])
    fp = jax.nn.relu(jnp.dot(fp, fp2W[...], preferred_element_type=jnp.float32) + fp2b[...])
    y = jax.nn.relu(jnp.dot(gf, h1W[0:256, :], preferred_element_type=jnp.float32)
                    + jnp.dot(fp, h1W[256:512, :], preferred_element_type=jnp.float32) + h1b[...])
    y = jax.nn.relu(jnp.dot(y, h2W[...], preferred_element_type=jnp.float32) + h2b[...])
    y = jax.nn.relu(jnp.dot(y, h3W[...], preferred_element_type=jnp.float32) + h3b[...])
    y = jnp.dot(y, h4W[...], preferred_element_type=jnp.float32) + h4b[...]
    out_ref[...] = jax.nn.sigmoid(y)


def _heads(g, finger_pad, p):
    fp1W_pad = jnp.pad(p['fp1W'], ((0, 47), (0, 0)))
    args = (g, finger_pad, p['fcg_W'], p['fcg_b'].reshape(1, -1), fp1W_pad,
            p['fp1b'].reshape(1, -1), p['fp2W'], p['fp2b'].reshape(1, -1),
            p['h1W'], p['h1b'].reshape(1, -1), p['h2W'], p['h2b'].reshape(1, -1),
            p['h3W'], p['h3b'].reshape(1, -1), p['h4W'], p['h4b'].reshape(1, -1))
    return pl.pallas_call(
        _heads_body,
        out_shape=jax.ShapeDtypeStruct((NUM_GRAPHS, 1), jnp.float32),
    )(*args)


def kernel(x, edge_attr, finger, params, edge_index, batch):
    p = params
    N = x.shape[0]
    src = edge_index[0]
    dst = edge_index[1]

    def gine(h, eW, eb, W1, b1, W2, b2):
        m = jax.nn.relu(h[src] + edge_attr @ eW + eb)
        agg = jax.ops.segment_sum(m, dst, num_segments=N)
        z = h + agg
        return jax.nn.relu(z @ W1 + b1) @ W2 + b2

    def bn(h, g, b):
        mu = jnp.mean(h, axis=0)
        var = jnp.var(h, axis=0)
        return g * (h - mu) / jnp.sqrt(var + 1e-5) + b

    h = jax.nn.relu(gine(x, p['e1W'], p['e1b'], p['n1W1'], p['n1b1'], p['n1W2'], p['n1b2']))
    h = bn(h, p['bn1g'], p['bn1b'])
    h = jax.nn.relu(gine(h, p['e2W'], p['e2b'], p['n2W1'], p['n2b1'], p['n2W2'], p['n2b2']))
    h = bn(h, p['bn2g'], p['bn2b'])
    h = jax.nn.relu(gine(h, p['e3W'], p['e3b'], p['n3W1'], p['n3b1'], p['n3W2'], p['n3b2']))
    h = bn(h, p['bn3g'], p['bn3b'])

    g = jax.ops.segment_sum(h, batch, num_segments=NUM_GRAPHS)
    finger_pad = jnp.pad(finger, ((0, 0), (0, 47)))
    return _heads(g, finger_pad, params)


# speedup vs baseline: 1.3432x; 1.3432x over previous
import functools

import jax
import jax.numpy as jnp
from jax import lax
from jax.experimental import pallas as pl
from jax.experimental.pallas import tpu as pltpu
from jax.experimental.pallas import tpu_sc as plsc

N_REAL = 10000
NP = 10240
NH = NP // 2
SR = NH + 128
E_REAL = 160000
EP = 163840
G = 128
BM = 256
NBLK = NP // BM
B = 256
NEB = EP // 16 // B


def _edge_sc(h4flat, proj, srcp, dstp, ncs):
    mesh = plsc.VectorSubcoreMesh(core_axis_name="c", subcore_axis_name="s")

    @functools.partial(
        pl.kernel,
        mesh=mesh,
        out_type=jax.ShapeDtypeStruct((ncs * NP, 128), jnp.float32),
        scratch_types=[
            pltpu.VMEM((B,), jnp.int32),
            pltpu.VMEM((B,), jnp.int32),
            pltpu.VMEM((B,), jnp.int32),
            pltpu.VMEM((B, 128), jnp.float32),
            pltpu.VMEM((B, 128), jnp.float32),
            pltpu.VMEM((164, 128), jnp.float32),
            pltpu.VMEM_SHARED((SR, 128), jnp.float32),
            pltpu.SemaphoreType.DMA,
        ],
    )
    def k(h4_hbm, proj_hbm, src_hbm, dst_hbm, out_hbm,
          srcv, dstv, idxv, gath, projv, zbuf, aggs, sem):
        core = lax.axis_index("c")
        tid = lax.axis_index("s")
        nbase = core * NH
        ebase0 = tid * (EP // 16)
        lanes = lax.iota(jnp.int32, 16)

        def zloop(i, carry):
            for j in range(8):
                zbuf[i, pl.ds(j * 16, 16)] = jnp.zeros((16,), jnp.float32)
            return carry
        lax.fori_loop(0, 164, zloop, 0)
        for t in range(2):
            pltpu.sync_copy(zbuf, aggs.at[pl.ds(tid * 328 + t * 164, 164)])
        plsc.subcore_barrier()

        for c in range(ncs):
            def blk_loop(blk, carry):
                ebase = ebase0 + blk * B
                pltpu.sync_copy(src_hbm.at[pl.ds(ebase, B)], srcv)
                pltpu.sync_copy(dst_hbm.at[pl.ds(ebase, B)], dstv)

                def remap(i, cr):
                    s = pl.ds(i * 16, 16)
                    d = dstv[s] - nbase
                    valid = (d >= 0) & (d < NH)
                    dmy = NH + lanes + (i % 8) * 16
                    dstv[s] = jnp.where(valid, d, dmy)
                    if c > 0:
                        idxv[s] = srcv[s] + (c * NP)
                    return cr
                lax.fori_loop(0, B // 16, remap, 0)

                if c == 0:
                    pltpu.async_copy(h4_hbm.at[srcv], gath, sem).wait()
                else:
                    pltpu.async_copy(h4_hbm.at[idxv], gath, sem).wait()
                pltpu.sync_copy(proj_hbm.at[pl.ds(c * EP + ebase, B)], projv)

                def row_loop(i, cr):
                    for j in range(8):
                        s = pl.ds(j * 16, 16)
                        gath[i, s] = jnp.maximum(gath[i, s] + projv[i, s], 0.0)
                    return cr
                lax.fori_loop(0, B, row_loop, 0)
                pltpu.sync_copy(gath, aggs.at[dstv], add=True)
                return carry
            lax.fori_loop(0, NEB, blk_loop, 0)
            plsc.subcore_barrier()
            obase = c * NP + nbase + tid * (NH // 16)
            pltpu.sync_copy(aggs.at[pl.ds(tid * (NH // 16), NH // 16)],
                            out_hbm.at[pl.ds(obase, NH // 16)])
            if c < ncs - 1:
                for t in range(2):
                    pltpu.sync_copy(zbuf, aggs.at[pl.ds(tid * 328 + t * 164, 164)])
            plsc.subcore_barrier()

    return k(h4flat, proj, srcp, dstp)


def _proj_body(ea_ref, w_ref, b_ref, o_ref):
    o_ref[...] = jnp.dot(ea_ref[...], w_ref[0],
                         preferred_element_type=jnp.float32) + b_ref[0]


def _proj(ea_pad, eW, eb, ncs):
    be = 2048
    ne = EP // be
    wp = jnp.pad(eW, ((0, 16 - eW.shape[0]), (0, ncs * 128 - eW.shape[1])))
    wp = wp.reshape(16, ncs, 128).transpose(1, 0, 2)
    bp = jnp.pad(eb, ((0, ncs * 128 - eb.shape[0]),)).reshape(ncs, 1, 128)
    return pl.pallas_call(
        _proj_body,
        grid=(ncs, ne),
        in_specs=[
            pl.BlockSpec((be, 16), lambda c, e: (e, 0)),
            pl.BlockSpec((1, 16, 128), lambda c, e: (c, 0, 0)),
            pl.BlockSpec((1, 1, 128), lambda c, e: (c, 0, 0)),
        ],
        out_specs=pl.BlockSpec((be, 128), lambda c, e: (c * ne + e, 0)),
        out_shape=jax.ShapeDtypeStruct((ncs * EP, 128), jnp.float32),
    )(ea_pad, wp, bp)


def _mlp_body(ncs, mid, h4_ref, agg_ref, w1_ref, b1_ref, w2_ref, b2_ref,
              h_ref, st_ref):
    i = pl.program_id(0)
    acc = jnp.zeros((BM, mid), jnp.float32)
    for c in range(ncs):
        zc = h4_ref[c] + agg_ref[c]
        acc += jnp.dot(zc, w1_ref[c * 128:(c + 1) * 128, :],
                       preferred_element_type=jnp.float32)
    y = jax.nn.relu(acc + b1_ref[...])
    h = jax.nn.relu(jnp.dot(y, w2_ref[...], preferred_element_type=jnp.float32)
                    + b2_ref[...])
    rows = lax.broadcasted_iota(jnp.int32, (BM, 1), 0) + i * BM
    hm = jnp.where(rows < N_REAL, h, 0.0)
    st_ref[0] = jnp.concatenate(
        [jnp.sum(hm, axis=0, keepdims=True),
         jnp.sum(hm * hm, axis=0, keepdims=True)], axis=0)
    h_ref[...] = h


def _mlp(h4, agg, W1p, b1, W2, b2, ncs, mid):
    return pl.pallas_call(
        functools.partial(_mlp_body, ncs, mid),
        grid=(NBLK,),
        in_specs=[
            pl.BlockSpec((ncs, BM, 128), lambda i: (0, i, 0)),
            pl.BlockSpec((ncs, BM, 128), lambda i: (0, i, 0)),
            pl.BlockSpec((ncs * 128, mid), lambda i: (0, 0)),
            pl.BlockSpec((1, mid), lambda i: (0, 0)),
            pl.BlockSpec((mid, 512), lambda i: (0, 0)),
            pl.BlockSpec((1, 512), lambda i: (0, 0)),
        ],
        out_specs=[
            pl.BlockSpec((BM, 512), lambda i: (i, 0)),
            pl.BlockSpec((1, 2, 512), lambda i: (i, 0, 0)),
        ],
        out_shape=[
            jax.ShapeDtypeStruct((NP, 512), jnp.float32),
            jax.ShapeDtypeStruct((NBLK, 2, 512), jnp.float32),
        ],
    )(h4, agg, W1p, b1.reshape(1, -1), W2, b2.reshape(1, -1))


def _bn_body(h_ref, st_ref, g_ref, b_ref, h4_ref):
    s = jnp.sum(st_ref[...], axis=0)
    mu = s[0:1, :] / N_REAL
    var = s[1:2, :] / N_REAL - mu * mu
    a = g_ref[...] * lax.rsqrt(var + 1e-5)
    cc = b_ref[...] - a * mu
    bnh = a * h_ref[...] + cc
    for c in range(4):
        h4_ref[c] = bnh[:, c * 128:(c + 1) * 128]


def _bn(h_raw, stats, gamma, beta):
    return pl.pallas_call(
        _bn_body,
        grid=(NBLK,),
        in_specs=[
            pl.BlockSpec((BM, 512), lambda i: (i, 0)),
            pl.BlockSpec((NBLK, 2, 512), lambda i: (0, 0, 0)),
            pl.BlockSpec((1, 512), lambda i: (0, 0)),
            pl.BlockSpec((1, 512), lambda i: (0, 0)),
        ],
        out_specs=pl.BlockSpec((4, BM, 128), lambda i: (0, i, 0)),
        out_shape=jax.ShapeDtypeStruct((4, NP, 128), jnp.float32),
    )(h_raw, stats, gamma.reshape(1, -1), beta.reshape(1, -1))


def _pool_body(h_ref, b_ref, g_ref, c_ref):
    i = pl.program_id(0)
    bt = b_ref[0, 0, :]
    oh = (lax.broadcasted_iota(jnp.int32, (G, BM), 0)
          == bt[None, :]).astype(jnp.float32)

    @pl.when(i == 0)
    def _():
        g_ref[...] = jnp.zeros_like(g_ref)
        c_ref[...] = jnp.zeros_like(c_ref)

    g_ref[...] += jnp.dot(oh, h_ref[...], preferred_element_type=jnp.float32)
    c_ref[...] += jnp.sum(oh, axis=1, keepdims=True)


def _pool(h_raw, batch3d):
    return pl.pallas_call(
        _pool_body,
        grid=(NBLK,),
        in_specs=[
            pl.BlockSpec((BM, 512), lambda i: (i, 0)),
            pl.BlockSpec((1, 1, BM), lambda i: (i, 0, 0)),
        ],
        out_specs=[
            pl.BlockSpec((G, 512), lambda i: (0, 0)),
            pl.BlockSpec((G, 1), lambda i: (0, 0)),
        ],
        out_shape=[
            jax.ShapeDtypeStruct((G, 512), jnp.float32),
            jax.ShapeDtypeStruct((G, 1), jnp.float32),
        ],
    )(h_raw, batch3d)


def _heads_body(g_ref, cnt_ref, st_ref, bg_ref, bb_ref, finger_ref,
                fcgW, fcgb, fp1W, fp1b, fp2W, fp2b,
                h1W, h1b, h2W, h2b, h3W, h3b, h4W, h4b, out_ref):
    s = jnp.sum(st_ref[...], axis=0)
    mu = s[0:1, :] / N_REAL
    var = s[1:2, :] / N_REAL - mu * mu
    a = bg_ref[...] * lax.rsqrt(var + 1e-5)
    cc = bb_ref[...] - a * mu
    gaff = g_ref[...] * a + cnt_ref[...] * cc
    gf = jax.nn.relu(jnp.dot(gaff, fcgW[...], preferred_element_type=jnp.float32) + fcgb[...])
    fp = jax.nn.relu(jnp.dot(finger_ref[...], fp1W[...], preferred_element_type=jnp.float32) + fp1b[...])
    fp = jax.nn.relu(jnp.dot(fp, fp2W[...], preferred_element_type=jnp.float32) + fp2b[...])
    y = jax.nn.relu(jnp.dot(gf, h1W[0:256, :], preferred_element_type=jnp.float32)
                    + jnp.dot(fp, h1W[256:512, :], preferred_element_type=jnp.float32)
                    + h1b[...])
    y = jax.nn.relu(jnp.dot(y, h2W[...], preferred_element_type=jnp.float32) + h2b[...])
    y = jax.nn.relu(jnp.dot(y, h3W[...], preferred_element_type=jnp.float32) + h3b[...])
    y = jnp.dot(y, h4W[...], preferred_element_type=jnp.float32) + h4b[...]
    out_ref[...] = jax.nn.sigmoid(y)


def _heads(gsum, cnt, stats3, p, finger_pad):
    fp1W_pad = jnp.pad(p['fp1W'], ((0, 47), (0, 0)))
    args = (gsum, cnt, stats3, p['bn3g'].reshape(1, -1), p['bn3b'].reshape(1, -1),
            finger_pad, p['fcg_W'], p['fcg_b'].reshape(1, -1), fp1W_pad,
            p['fp1b'].reshape(1, -1), p['fp2W'], p['fp2b'].reshape(1, -1),
            p['h1W'], p['h1b'].reshape(1, -1), p['h2W'], p['h2b'].reshape(1, -1),
            p['h3W'], p['h3b'].reshape(1, -1), p['h4W'], p['h4b'].reshape(1, -1))
    return pl.pallas_call(
        _heads_body,
        out_shape=jax.ShapeDtypeStruct((G, 1), jnp.float32),
    )(*args)


def kernel(x, edge_attr, finger, params, edge_index, batch):
    p = params
    src = edge_index[0]
    dst = edge_index[1]

    npad = EP - E_REAL
    src_p = jnp.concatenate([src, jnp.arange(npad, dtype=jnp.int32) % N_REAL])
    dst_p = jnp.concatenate([dst, N_REAL + (jnp.arange(npad, dtype=jnp.int32) % (NP - N_REAL))])
    ea_pad = jnp.pad(edge_attr, ((0, npad), (0, 6)))
    x_pad = jnp.pad(x, ((0, NP - N_REAL), (0, 50)))
    batch3d = jnp.pad(batch, ((0, NP - N_REAL),), constant_values=G).reshape(NBLK, 1, BM)
    finger_pad = jnp.pad(finger, ((0, 0), (0, 47)))

    proj1 = _proj(ea_pad, p['e1W'], p['e1b'], 1)
    agg1 = _edge_sc(x_pad, proj1, src_p, dst_p, 1)
    W1p = jnp.pad(p['n1W1'], ((0, 50), (0, 0)))
    h1_raw, st1 = _mlp(x_pad[None], agg1.reshape(1, NP, 128), W1p,
                       p['n1b1'], p['n1W2'], p['n1b2'], 1, 256)
    h1c = _bn(h1_raw, st1, p['bn1g'], p['bn1b'])

    proj2 = _proj(ea_pad, p['e2W'], p['e2b'], 4)
    agg2 = _edge_sc(h1c.reshape(4 * NP, 128), proj2, src_p, dst_p, 4)
    h2_raw, st2 = _mlp(h1c, agg2.reshape(4, NP, 128), p['n2W1'],
                       p['n2b1'], p['n2W2'], p['n2b2'], 4, 512)
    h2c = _bn(h2_raw, st2, p['bn2g'], p['bn2b'])

    proj3 = _proj(ea_pad, p['e3W'], p['e3b'], 4)
    agg3 = _edge_sc(h2c.reshape(4 * NP, 128), proj3, src_p, dst_p, 4)
    h3_raw, st3 = _mlp(h2c, agg3.reshape(4, NP, 128), p['n3W1'],
                       p['n3b1'], p['n3W2'], p['n3b2'], 4, 512)

    gsum, cnt = _pool(h3_raw, batch3d)
    return _heads(gsum, cnt, st3, p, finger_pad)


# speedup vs baseline: 1.7809x; 1.3258x over previous
import functools

import jax
import jax.numpy as jnp
from jax import lax
from jax.experimental import pallas as pl
from jax.experimental.pallas import tpu as pltpu
from jax.experimental.pallas import tpu_sc as plsc

N_REAL = 10000
NP = 10240
NH = NP // 2
SR = NH + 128
E_REAL = 160000
EP = 163840
G = 128
BM = 256
NBLK = NP // BM
B = 128
NEB = EP // 16 // B


def _edge_sc(h4flat, proj, src3, dst2, ncs):
    mesh = plsc.VectorSubcoreMesh(core_axis_name="c", subcore_axis_name="s")

    @functools.partial(
        pl.kernel,
        mesh=mesh,
        out_type=jax.ShapeDtypeStruct((ncs * NP, 128), jnp.float32),
        scratch_types=[
            pltpu.VMEM((B,), jnp.int32),
            pltpu.VMEM((B,), jnp.int32),
            pltpu.VMEM((B,), jnp.int32),
            pltpu.VMEM((B,), jnp.int32),
            pltpu.VMEM((B, 128), jnp.float32),
            pltpu.VMEM((B, 128), jnp.float32),
            pltpu.VMEM((B, 128), jnp.float32),
            pltpu.VMEM((B, 128), jnp.float32),
            pltpu.VMEM((160, 128), jnp.float32),
            pltpu.VMEM_SHARED((SR, 128), jnp.float32),
            pltpu.SemaphoreType.DMA,
            pltpu.SemaphoreType.DMA,
            pltpu.SemaphoreType.DMA,
            pltpu.SemaphoreType.DMA,
        ],
    )
    def k(h4_hbm, proj_hbm, src3_hbm, dst2_hbm, out_hbm,
          srcv0, srcv1, dstv0, dstv1, gath0, gath1, projv0, projv1,
          zbuf, aggs, gs0, gs1, ps0, ps1):
        core = lax.axis_index("c")
        tid = lax.axis_index("s")
        srcvs = (srcv0, srcv1)
        dstvs = (dstv0, dstv1)
        gaths = (gath0, gath1)
        projvs = (projv0, projv1)
        gsems = (gs0, gs1)
        psems = (ps0, ps1)

        def zloop(i, carry):
            for j in range(8):
                zbuf[i, pl.ds(j * 16, 16)] = jnp.zeros((16,), jnp.float32)
            return carry
        lax.fori_loop(0, 160, zloop, 0)

        def zero_stripes():
            for t in range(2):
                pltpu.sync_copy(zbuf, aggs.at[pl.ds(tid * 320 + t * 160, 160)])
            pltpu.sync_copy(zbuf.at[pl.ds(0, 8)],
                            aggs.at[pl.ds(NH + tid * 8, 8)])
        zero_stripes()
        plsc.subcore_barrier()

        for c in range(ncs):
            pbase = c * EP + tid * (EP // 16)

            def ld_idx(blk, b):
                pltpu.sync_copy(src3_hbm.at[tid, blk], srcvs[b])
                pltpu.sync_copy(dst2_hbm.at[core, tid, blk], dstvs[b])
                if c > 0:
                    def ao(i, cr):
                        s = pl.ds(i * 16, 16)
                        srcvs[b][s] = srcvs[b][s] + (c * NP)
                        return cr
                    lax.fori_loop(0, B // 16, ao, 0)

            def issue(blk, b):
                pltpu.async_copy(h4_hbm.at[srcvs[b]], gaths[b], gsems[b])
                pltpu.async_copy(proj_hbm.at[pl.ds(pbase + blk * B, B)],
                                 projvs[b], psems[b])

            def work(blk, b, do_issue):
                pltpu.make_async_copy(
                    h4_hbm.at[srcvs[b]], gaths[b], gsems[b]).wait()
                pltpu.make_async_copy(
                    proj_hbm.at[pl.ds(pbase + blk * B, B)],
                    projvs[b], psems[b]).wait()

                def row_loop(i, cr):
                    for j in range(8):
                        s = pl.ds(j * 16, 16)
                        gaths[b][i, s] = jnp.maximum(
                            gaths[b][i, s] + projvs[b][i, s], 0.0)
                    return cr
                lax.fori_loop(0, B, row_loop, 0)
                pltpu.sync_copy(gaths[b], aggs.at[dstvs[b]], add=True)
                if do_issue:
                    ld_idx(blk + 2, b)
                    issue(blk + 2, b)

            for b in range(2):
                ld_idx(b, b)
                issue(b, b)

            def pair_body(it, carry):
                for b in range(2):
                    work(it * 2 + b, b, True)
                return carry
            lax.fori_loop(0, NEB // 2 - 1, pair_body, 0)
            for b in range(2):
                work(NEB - 2 + b, b, False)

            plsc.subcore_barrier()
            obase = c * NP + core * NH + tid * (NH // 16)
            pltpu.sync_copy(aggs.at[pl.ds(tid * (NH // 16), NH // 16)],
                            out_hbm.at[pl.ds(obase, NH // 16)])
            if c < ncs - 1:
                zero_stripes()
            plsc.subcore_barrier()

    return k(h4flat, proj, src3, dst2)


def _proj_body(ea_ref, w_ref, b_ref, o_ref):
    o_ref[...] = jnp.dot(ea_ref[...], w_ref[0],
                         preferred_element_type=jnp.float32) + b_ref[0]


def _proj(ea_pad, eW, eb, ncs):
    be = 2048
    ne = EP // be
    wp = jnp.pad(eW, ((0, 16 - eW.shape[0]), (0, ncs * 128 - eW.shape[1])))
    wp = wp.reshape(16, ncs, 128).transpose(1, 0, 2)
    bp = jnp.pad(eb, ((0, ncs * 128 - eb.shape[0]),)).reshape(ncs, 1, 128)
    return pl.pallas_call(
        _proj_body,
        grid=(ncs, ne),
        in_specs=[
            pl.BlockSpec((be, 16), lambda c, e: (e, 0)),
            pl.BlockSpec((1, 16, 128), lambda c, e: (c, 0, 0)),
            pl.BlockSpec((1, 1, 128), lambda c, e: (c, 0, 0)),
        ],
        out_specs=pl.BlockSpec((be, 128), lambda c, e: (c * ne + e, 0)),
        out_shape=jax.ShapeDtypeStruct((ncs * EP, 128), jnp.float32),
    )(ea_pad, wp, bp)


def _mlp_body(ncs, mid, h4_ref, agg_ref, w1_ref, b1_ref, w2_ref, b2_ref,
              h_ref, st_ref, z_ref):
    i = pl.program_id(0)
    for c in range(ncs):
        z_ref[:, c * 128:(c + 1) * 128] = h4_ref[c] + agg_ref[c]
    acc = jnp.dot(z_ref[...], w1_ref[...], preferred_element_type=jnp.float32)
    y = jax.nn.relu(acc + b1_ref[...])
    h = jax.nn.relu(jnp.dot(y, w2_ref[...], preferred_element_type=jnp.float32)
                    + b2_ref[...])
    rows = lax.broadcasted_iota(jnp.int32, (BM, 1), 0) + i * BM
    hm = jnp.where(rows < N_REAL, h, 0.0)
    st_ref[0] = jnp.concatenate(
        [jnp.sum(hm, axis=0, keepdims=True),
         jnp.sum(hm * hm, axis=0, keepdims=True)], axis=0)
    h_ref[...] = h


def _mlp(h4, agg, W1p, b1, W2, b2, ncs, mid):
    return pl.pallas_call(
        functools.partial(_mlp_body, ncs, mid),
        grid=(NBLK,),
        in_specs=[
            pl.BlockSpec((ncs, BM, 128), lambda i: (0, i, 0)),
            pl.BlockSpec((ncs, BM, 128), lambda i: (0, i, 0)),
            pl.BlockSpec((ncs * 128, mid), lambda i: (0, 0)),
            pl.BlockSpec((1, mid), lambda i: (0, 0)),
            pl.BlockSpec((mid, 512), lambda i: (0, 0)),
            pl.BlockSpec((1, 512), lambda i: (0, 0)),
        ],
        out_specs=[
            pl.BlockSpec((BM, 512), lambda i: (i, 0)),
            pl.BlockSpec((1, 2, 512), lambda i: (i, 0, 0)),
        ],
        out_shape=[
            jax.ShapeDtypeStruct((NP, 512), jnp.float32),
            jax.ShapeDtypeStruct((NBLK, 2, 512), jnp.float32),
        ],
        scratch_shapes=[pltpu.VMEM((BM, ncs * 128), jnp.float32)],
    )(h4, agg, W1p, b1.reshape(1, -1), W2, b2.reshape(1, -1))


def _bn_body(h_ref, st_ref, g_ref, b_ref, h4_ref):
    s = jnp.sum(st_ref[...], axis=0)
    mu = s[0:1, :] / N_REAL
    var = s[1:2, :] / N_REAL - mu * mu
    a = g_ref[...] * lax.rsqrt(var + 1e-5)
    cc = b_ref[...] - a * mu
    bnh = a * h_ref[...] + cc
    for c in range(4):
        h4_ref[c] = bnh[:, c * 128:(c + 1) * 128]


def _bn(h_raw, stats, gamma, beta):
    return pl.pallas_call(
        _bn_body,
        grid=(NBLK,),
        in_specs=[
            pl.BlockSpec((BM, 512), lambda i: (i, 0)),
            pl.BlockSpec((NBLK, 2, 512), lambda i: (0, 0, 0)),
            pl.BlockSpec((1, 512), lambda i: (0, 0)),
            pl.BlockSpec((1, 512), lambda i: (0, 0)),
        ],
        out_specs=pl.BlockSpec((4, BM, 128), lambda i: (0, i, 0)),
        out_shape=jax.ShapeDtypeStruct((4, NP, 128), jnp.float32),
    )(h_raw, stats, gamma.reshape(1, -1), beta.reshape(1, -1))


def _pool_body(h_ref, b_ref, g_ref, c_ref):
    i = pl.program_id(0)
    bt = b_ref[0, 0, :]
    oh = (lax.broadcasted_iota(jnp.int32, (G, BM), 0)
          == bt[None, :]).astype(jnp.float32)

    @pl.when(i == 0)
    def _():
        g_ref[...] = jnp.zeros_like(g_ref)
        c_ref[...] = jnp.zeros_like(c_ref)

    g_ref[...] += jnp.dot(oh, h_ref[...], preferred_element_type=jnp.float32,
                          precision=lax.Precision.HIGHEST)
    c_ref[...] += jnp.sum(oh, axis=1, keepdims=True)


def _pool(h_raw, batch3d):
    return pl.pallas_call(
        _pool_body,
        grid=(NBLK,),
        in_specs=[
            pl.BlockSpec((BM, 512), lambda i: (i, 0)),
            pl.BlockSpec((1, 1, BM), lambda i: (i, 0, 0)),
        ],
        out_specs=[
            pl.BlockSpec((G, 512), lambda i: (0, 0)),
            pl.BlockSpec((G, 1), lambda i: (0, 0)),
        ],
        out_shape=[
            jax.ShapeDtypeStruct((G, 512), jnp.float32),
            jax.ShapeDtypeStruct((G, 1), jnp.float32),
        ],
    )(h_raw, batch3d)


def _heads_body(g_ref, cnt_ref, st_ref, bg_ref, bb_ref, finger_ref,
                fcgW, fcgb, fp1W, fp1b, fp2W, fp2b,
                h1W, h1b, h2W, h2b, h3W, h3b, h4W, h4b, out_ref):
    s = jnp.sum(st_ref[...], axis=0)
    mu = s[0:1, :] / N_REAL
    var = s[1:2, :] / N_REAL - mu * mu
    a = bg_ref[...] * lax.rsqrt(var + 1e-5)
    cc = bb_ref[...] - a * mu
    gaff = g_ref[...] * a + cnt_ref[...] * cc
    gf = jax.nn.relu(jnp.dot(gaff, fcgW[...], preferred_element_type=jnp.float32) + fcgb[...])
    fp = jax.nn.relu(jnp.dot(finger_ref[...], fp1W[...], preferred_element_type=jnp.float32) + fp1b[...])
    fp = jax.nn.relu(jnp.dot(fp, fp2W[...], preferred_element_type=jnp.float32) + fp2b[...])
    xc = jnp.concatenate([gf, fp], axis=1)
    y = jax.nn.relu(jnp.dot(xc, h1W[...], preferred_element_type=jnp.float32)
                    + h1b[...])
    y = jax.nn.relu(jnp.dot(y, h2W[...], preferred_element_type=jnp.float32) + h2b[...])
    y = jax.nn.relu(jnp.dot(y, h3W[...], preferred_element_type=jnp.float32) + h3b[...])
    y = jnp.dot(y, h4W[...], preferred_element_type=jnp.float32) + h4b[...]
    out_ref[...] = jax.nn.sigmoid(y)


def _heads(gsum, cnt, stats3, p, finger_pad):
    fp1W_pad = jnp.pad(p['fp1W'], ((0, 47), (0, 0)))
    args = (gsum, cnt, stats3, p['bn3g'].reshape(1, -1), p['bn3b'].reshape(1, -1),
            finger_pad, p['fcg_W'], p['fcg_b'].reshape(1, -1), fp1W_pad,
            p['fp1b'].reshape(1, -1), p['fp2W'], p['fp2b'].reshape(1, -1),
            p['h1W'], p['h1b'].reshape(1, -1), p['h2W'], p['h2b'].reshape(1, -1),
            p['h3W'], p['h3b'].reshape(1, -1), p['h4W'], p['h4b'].reshape(1, -1))
    return pl.pallas_call(
        _heads_body,
        out_shape=jax.ShapeDtypeStruct((G, 1), jnp.float32),
    )(*args)


def kernel(x, edge_attr, finger, params, edge_index, batch):
    p = params
    src = edge_index[0]
    dst = edge_index[1]

    npad = EP - E_REAL
    src_p = jnp.concatenate([src, jnp.arange(npad, dtype=jnp.int32) % N_REAL])
    dst_p = jnp.concatenate([dst, N_REAL + (jnp.arange(npad, dtype=jnp.int32) % (NP - N_REAL))])
    src3 = src_p.reshape(16, NEB, B)
    spread = NH + (jnp.arange(EP, dtype=jnp.int32) % 128)
    d0 = jnp.where(dst_p < NH, dst_p, spread)
    d1 = jnp.where(dst_p >= NH, dst_p - NH, spread)
    dst2 = jnp.stack([d0, d1]).reshape(2, 16, NEB, B)
    ea_pad = jnp.pad(edge_attr, ((0, npad), (0, 6)))
    x_pad = jnp.pad(x, ((0, NP - N_REAL), (0, 50)))
    batch3d = jnp.pad(batch, ((0, NP - N_REAL),), constant_values=G).reshape(NBLK, 1, BM)
    finger_pad = jnp.pad(finger, ((0, 0), (0, 47)))

    proj1 = _proj(ea_pad, p['e1W'], p['e1b'], 1)
    agg1 = _edge_sc(x_pad, proj1, src3, dst2, 1)
    W1p = jnp.pad(p['n1W1'], ((0, 50), (0, 0)))
    h1_raw, st1 = _mlp(x_pad[None], agg1.reshape(1, NP, 128), W1p,
                       p['n1b1'], p['n1W2'], p['n1b2'], 1, 256)
    h1c = _bn(h1_raw, st1, p['bn1g'], p['bn1b'])

    proj2 = _proj(ea_pad, p['e2W'], p['e2b'], 4)
    agg2 = _edge_sc(h1c.reshape(4 * NP, 128), proj2, src3, dst2, 4)
    h2_raw, st2 = _mlp(h1c, agg2.reshape(4, NP, 128), p['n2W1'],
                       p['n2b1'], p['n2W2'], p['n2b2'], 4, 512)
    h2c = _bn(h2_raw, st2, p['bn2g'], p['bn2b'])

    proj3 = _proj(ea_pad, p['e3W'], p['e3b'], 4)
    agg3 = _edge_sc(h2c.reshape(4 * NP, 128), proj3, src3, dst2, 4)
    h3_raw, st3 = _mlp(h2c, agg3.reshape(4, NP, 128), p['n3W1'],
                       p['n3b1'], p['n3W2'], p['n3b2'], 4, 512)

    gsum, cnt = _pool(h3_raw, batch3d)
    return _heads(gsum, cnt, st3, p, finger_pad)


# speedup vs baseline: 1.9686x; 1.1054x over previous
import functools

import jax
import jax.numpy as jnp
from jax import lax
from jax.experimental import pallas as pl
from jax.experimental.pallas import tpu as pltpu
from jax.experimental.pallas import tpu_sc as plsc

N_REAL = 10000
NP = 10240
NH = NP // 2
SR = NH + 128
E_REAL = 160000
EP = 163840
G = 128
BM = 256
NBLK = NP // BM
B = 128
NEB = EP // 16 // B


def _edge_sc(h4flat, proj, src3, dst2, ncs):
    mesh = plsc.VectorSubcoreMesh(core_axis_name="c", subcore_axis_name="s")

    @functools.partial(
        pl.kernel,
        mesh=mesh,
        out_type=jax.ShapeDtypeStruct((ncs * NP, 128), jnp.float32),
        scratch_types=[
            pltpu.VMEM((B,), jnp.int32),
            pltpu.VMEM((B,), jnp.int32),
            pltpu.VMEM((B,), jnp.int32),
            pltpu.VMEM((B,), jnp.int32),
            pltpu.VMEM((B, 128), jnp.float32),
            pltpu.VMEM((B, 128), jnp.float32),
            pltpu.VMEM((B, 128), jnp.float32),
            pltpu.VMEM((B, 128), jnp.float32),
            pltpu.VMEM((160, 128), jnp.float32),
            pltpu.VMEM_SHARED((SR, 128), jnp.float32),
            pltpu.SemaphoreType.DMA,
            pltpu.SemaphoreType.DMA,
            pltpu.SemaphoreType.DMA,
            pltpu.SemaphoreType.DMA,
            pltpu.SemaphoreType.DMA,
            pltpu.SemaphoreType.DMA,
        ],
    )
    def k(h4_hbm, proj_hbm, src3_hbm, dst2_hbm, out_hbm,
          srcv0, srcv1, dstv0, dstv1, gath0, gath1, projv0, projv1,
          zbuf, aggs, gs0, gs1, ps0, ps1, ss0, ss1):
        ssems = (ss0, ss1)
        core = lax.axis_index("c")
        tid = lax.axis_index("s")
        srcvs = (srcv0, srcv1)
        dstvs = (dstv0, dstv1)
        gaths = (gath0, gath1)
        projvs = (projv0, projv1)
        gsems = (gs0, gs1)
        psems = (ps0, ps1)

        def zloop(i, carry):
            for j in range(8):
                zbuf[i, pl.ds(j * 16, 16)] = jnp.zeros((16,), jnp.float32)
            return carry
        lax.fori_loop(0, 160, zloop, 0)

        def zero_stripes():
            for t in range(2):
                pltpu.sync_copy(zbuf, aggs.at[pl.ds(tid * 320 + t * 160, 160)])
            pltpu.sync_copy(zbuf.at[pl.ds(0, 8)],
                            aggs.at[pl.ds(NH + tid * 8, 8)])
        zero_stripes()
        plsc.subcore_barrier()

        for c in range(ncs):
            pbase = c * EP + tid * (EP // 16)

            def ld_idx(blk, b):
                pltpu.sync_copy(src3_hbm.at[tid, blk], srcvs[b])
                pltpu.sync_copy(dst2_hbm.at[core, tid, blk], dstvs[b])
                if c > 0:
                    def ao(i, cr):
                        s = pl.ds(i * 16, 16)
                        srcvs[b][s] = srcvs[b][s] + (c * NP)
                        return cr
                    lax.fori_loop(0, B // 16, ao, 0)

            def issue(blk, b):
                pltpu.async_copy(h4_hbm.at[srcvs[b]], gaths[b], gsems[b])
                pltpu.async_copy(proj_hbm.at[pl.ds(pbase + blk * B, B)],
                                 projvs[b], psems[b])

            def work(blk, b, do_issue):
                pltpu.make_async_copy(
                    h4_hbm.at[srcvs[b]], gaths[b], gsems[b]).wait()
                pltpu.make_async_copy(
                    proj_hbm.at[pl.ds(pbase + blk * B, B)],
                    projvs[b], psems[b]).wait()

                def row_loop(i, cr):
                    for j in range(8):
                        s = pl.ds(j * 16, 16)
                        gaths[b][i, s] = jnp.maximum(
                            gaths[b][i, s] + projvs[b][i, s], 0.0)
                    return cr
                lax.fori_loop(0, B, row_loop, 0)
                pltpu.async_copy(gaths[b], aggs.at[dstvs[b]], ssems[b],
                                 add=True)
                if do_issue:
                    ld_idx(blk + 2, b)
                    pltpu.async_copy(
                        proj_hbm.at[pl.ds(pbase + (blk + 2) * B, B)],
                        projvs[b], psems[b])
                    pltpu.make_async_copy(
                        gaths[b], aggs.at[dstvs[b]], ssems[b]).wait()
                    pltpu.async_copy(h4_hbm.at[srcvs[b]], gaths[b], gsems[b])
                else:
                    pltpu.make_async_copy(
                        gaths[b], aggs.at[dstvs[b]], ssems[b]).wait()

            for b in range(2):
                ld_idx(b, b)
                issue(b, b)

            def pair_body(it, carry):
                for b in range(2):
                    work(it * 2 + b, b, True)
                return carry
            lax.fori_loop(0, NEB // 2 - 1, pair_body, 0)
            for b in range(2):
                work(NEB - 2 + b, b, False)

            plsc.subcore_barrier()
            obase = c * NP + core * NH + tid * (NH // 16)
            pltpu.sync_copy(aggs.at[pl.ds(tid * (NH // 16), NH // 16)],
                            out_hbm.at[pl.ds(obase, NH // 16)])
            if c < ncs - 1:
                zero_stripes()
            plsc.subcore_barrier()

    return k(h4flat, proj, src3, dst2)


def _proj_body(ea_ref, w_ref, b_ref, o_ref):
    o_ref[...] = jnp.dot(ea_ref[...], w_ref[0],
                         preferred_element_type=jnp.float32) + b_ref[0]


def _proj(ea_pad, eW, eb, ncs):
    be = 2048
    ne = EP // be
    wp = jnp.pad(eW, ((0, 16 - eW.shape[0]), (0, ncs * 128 - eW.shape[1])))
    wp = wp.reshape(16, ncs, 128).transpose(1, 0, 2)
    bp = jnp.pad(eb, ((0, ncs * 128 - eb.shape[0]),)).reshape(ncs, 1, 128)
    return pl.pallas_call(
        _proj_body,
        grid=(ncs, ne),
        in_specs=[
            pl.BlockSpec((be, 16), lambda c, e: (e, 0)),
            pl.BlockSpec((1, 16, 128), lambda c, e: (c, 0, 0)),
            pl.BlockSpec((1, 1, 128), lambda c, e: (c, 0, 0)),
        ],
        out_specs=pl.BlockSpec((be, 128), lambda c, e: (c * ne + e, 0)),
        out_shape=jax.ShapeDtypeStruct((ncs * EP, 128), jnp.float32),
    )(ea_pad, wp, bp)


def _mlp_body(ncs, mid, h4_ref, agg_ref, w1_ref, b1_ref, w2_ref, b2_ref,
              h_ref, st_ref, z_ref):
    i = pl.program_id(0)
    for c in range(ncs):
        z_ref[:, c * 128:(c + 1) * 128] = h4_ref[c] + agg_ref[c]
    acc = jnp.dot(z_ref[...], w1_ref[...], preferred_element_type=jnp.float32)
    y = jax.nn.relu(acc + b1_ref[...])
    h = jax.nn.relu(jnp.dot(y, w2_ref[...], preferred_element_type=jnp.float32)
                    + b2_ref[...])
    rows = lax.broadcasted_iota(jnp.int32, (BM, 1), 0) + i * BM
    hm = jnp.where(rows < N_REAL, h, 0.0)
    st_ref[0] = jnp.concatenate(
        [jnp.sum(hm, axis=0, keepdims=True),
         jnp.sum(hm * hm, axis=0, keepdims=True)], axis=0)
    h_ref[...] = h


def _mlp(h4, agg, W1p, b1, W2, b2, ncs, mid):
    return pl.pallas_call(
        functools.partial(_mlp_body, ncs, mid),
        grid=(NBLK,),
        in_specs=[
            pl.BlockSpec((ncs, BM, 128), lambda i: (0, i, 0)),
            pl.BlockSpec((ncs, BM, 128), lambda i: (0, i, 0)),
            pl.BlockSpec((ncs * 128, mid), lambda i: (0, 0)),
            pl.BlockSpec((1, mid), lambda i: (0, 0)),
            pl.BlockSpec((mid, 512), lambda i: (0, 0)),
            pl.BlockSpec((1, 512), lambda i: (0, 0)),
        ],
        out_specs=[
            pl.BlockSpec((BM, 512), lambda i: (i, 0)),
            pl.BlockSpec((1, 2, 512), lambda i: (i, 0, 0)),
        ],
        out_shape=[
            jax.ShapeDtypeStruct((NP, 512), jnp.float32),
            jax.ShapeDtypeStruct((NBLK, 2, 512), jnp.float32),
        ],
        scratch_shapes=[pltpu.VMEM((BM, ncs * 128), jnp.float32)],
    )(h4, agg, W1p, b1.reshape(1, -1), W2, b2.reshape(1, -1))


def _bn_body(h_ref, st_ref, g_ref, b_ref, h4_ref):
    s = jnp.sum(st_ref[...], axis=0)
    mu = s[0:1, :] / N_REAL
    var = s[1:2, :] / N_REAL - mu * mu
    a = g_ref[...] * lax.rsqrt(var + 1e-5)
    cc = b_ref[...] - a * mu
    bnh = a * h_ref[...] + cc
    for c in range(4):
        h4_ref[c] = bnh[:, c * 128:(c + 1) * 128]


def _bn(h_raw, stats, gamma, beta):
    return pl.pallas_call(
        _bn_body,
        grid=(NBLK,),
        in_specs=[
            pl.BlockSpec((BM, 512), lambda i: (i, 0)),
            pl.BlockSpec((NBLK, 2, 512), lambda i: (0, 0, 0)),
            pl.BlockSpec((1, 512), lambda i: (0, 0)),
            pl.BlockSpec((1, 512), lambda i: (0, 0)),
        ],
        out_specs=pl.BlockSpec((4, BM, 128), lambda i: (0, i, 0)),
        out_shape=jax.ShapeDtypeStruct((4, NP, 128), jnp.float32),
    )(h_raw, stats, gamma.reshape(1, -1), beta.reshape(1, -1))


def _pool_body(h_ref, b_ref, g_ref, c_ref):
    i = pl.program_id(0)
    bt = b_ref[0, 0, :]
    oh = (lax.broadcasted_iota(jnp.int32, (G, BM), 0)
          == bt[None, :]).astype(jnp.float32)

    @pl.when(i == 0)
    def _():
        g_ref[...] = jnp.zeros_like(g_ref)
        c_ref[...] = jnp.zeros_like(c_ref)

    g_ref[...] += jnp.dot(oh, h_ref[...], preferred_element_type=jnp.float32,
                          precision=lax.Precision.HIGHEST)
    c_ref[...] += jnp.sum(oh, axis=1, keepdims=True)


def _pool(h_raw, batch3d):
    return pl.pallas_call(
        _pool_body,
        grid=(NBLK,),
        in_specs=[
            pl.BlockSpec((BM, 512), lambda i: (i, 0)),
            pl.BlockSpec((1, 1, BM), lambda i: (i, 0, 0)),
        ],
        out_specs=[
            pl.BlockSpec((G, 512), lambda i: (0, 0)),
            pl.BlockSpec((G, 1), lambda i: (0, 0)),
        ],
        out_shape=[
            jax.ShapeDtypeStruct((G, 512), jnp.float32),
            jax.ShapeDtypeStruct((G, 1), jnp.float32),
        ],
    )(h_raw, batch3d)


def _heads_body(g_ref, cnt_ref, st_ref, bg_ref, bb_ref, finger_ref,
                fcgW, fcgb, fp1W, fp1b, fp2W, fp2b,
                h1W, h1b, h2W, h2b, h3W, h3b, h4W, h4b, out_ref):
    s = jnp.sum(st_ref[...], axis=0)
    mu = s[0:1, :] / N_REAL
    var = s[1:2, :] / N_REAL - mu * mu
    a = bg_ref[...] * lax.rsqrt(var + 1e-5)
    cc = bb_ref[...] - a * mu
    gaff = g_ref[...] * a + cnt_ref[...] * cc
    gf = jax.nn.relu(jnp.dot(gaff, fcgW[...], preferred_element_type=jnp.float32) + fcgb[...])
    fp = jax.nn.relu(jnp.dot(finger_ref[...], fp1W[...], preferred_element_type=jnp.float32) + fp1b[...])
    fp = jax.nn.relu(jnp.dot(fp, fp2W[...], preferred_element_type=jnp.float32) + fp2b[...])
    xc = jnp.concatenate([gf, fp], axis=1)
    y = jax.nn.relu(jnp.dot(xc, h1W[...], preferred_element_type=jnp.float32)
                    + h1b[...])
    y = jax.nn.relu(jnp.dot(y, h2W[...], preferred_element_type=jnp.float32) + h2b[...])
    y = jax.nn.relu(jnp.dot(y, h3W[...], preferred_element_type=jnp.float32) + h3b[...])
    y = jnp.dot(y, h4W[...], preferred_element_type=jnp.float32) + h4b[...]
    out_ref[...] = jax.nn.sigmoid(y)


def _heads(gsum, cnt, stats3, p, finger_pad):
    fp1W_pad = jnp.pad(p['fp1W'], ((0, 47), (0, 0)))
    args = (gsum, cnt, stats3, p['bn3g'].reshape(1, -1), p['bn3b'].reshape(1, -1),
            finger_pad, p['fcg_W'], p['fcg_b'].reshape(1, -1), fp1W_pad,
            p['fp1b'].reshape(1, -1), p['fp2W'], p['fp2b'].reshape(1, -1),
            p['h1W'], p['h1b'].reshape(1, -1), p['h2W'], p['h2b'].reshape(1, -1),
            p['h3W'], p['h3b'].reshape(1, -1), p['h4W'], p['h4b'].reshape(1, -1))
    return pl.pallas_call(
        _heads_body,
        out_shape=jax.ShapeDtypeStruct((G, 1), jnp.float32),
    )(*args)


def kernel(x, edge_attr, finger, params, edge_index, batch):
    p = params
    src = edge_index[0]
    dst = edge_index[1]

    npad = EP - E_REAL
    src_p = jnp.concatenate([src, jnp.arange(npad, dtype=jnp.int32) % N_REAL])
    dst_p = jnp.concatenate([dst, N_REAL + (jnp.arange(npad, dtype=jnp.int32) % (NP - N_REAL))])
    src3 = src_p.reshape(16, NEB, B)
    spread = NH + (jnp.arange(EP, dtype=jnp.int32) % 128)
    d0 = jnp.where(dst_p < NH, dst_p, spread)
    d1 = jnp.where(dst_p >= NH, dst_p - NH, spread)
    dst2 = jnp.stack([d0, d1]).reshape(2, 16, NEB, B)
    ea_pad = jnp.pad(edge_attr, ((0, npad), (0, 6)))
    x_pad = jnp.pad(x, ((0, NP - N_REAL), (0, 50)))
    batch3d = jnp.pad(batch, ((0, NP - N_REAL),), constant_values=G).reshape(NBLK, 1, BM)
    finger_pad = jnp.pad(finger, ((0, 0), (0, 47)))

    proj1 = _proj(ea_pad, p['e1W'], p['e1b'], 1)
    agg1 = _edge_sc(x_pad, proj1, src3, dst2, 1)
    W1p = jnp.pad(p['n1W1'], ((0, 50), (0, 0)))
    h1_raw, st1 = _mlp(x_pad[None], agg1.reshape(1, NP, 128), W1p,
                       p['n1b1'], p['n1W2'], p['n1b2'], 1, 256)
    h1c = _bn(h1_raw, st1, p['bn1g'], p['bn1b'])

    proj2 = _proj(ea_pad, p['e2W'], p['e2b'], 4)
    agg2 = _edge_sc(h1c.reshape(4 * NP, 128), proj2, src3, dst2, 4)
    h2_raw, st2 = _mlp(h1c, agg2.reshape(4, NP, 128), p['n2W1'],
                       p['n2b1'], p['n2W2'], p['n2b2'], 4, 512)
    h2c = _bn(h2_raw, st2, p['bn2g'], p['bn2b'])

    proj3 = _proj(ea_pad, p['e3W'], p['e3b'], 4)
    agg3 = _edge_sc(h2c.reshape(4 * NP, 128), proj3, src3, dst2, 4)
    h3_raw, st3 = _mlp(h2c, agg3.reshape(4, NP, 128), p['n3W1'],
                       p['n3b1'], p['n3W2'], p['n3b2'], 4, 512)

    gsum, cnt = _pool(h3_raw, batch3d)
    return _heads(gsum, cnt, st3, p, finger_pad)
